# baseline (speedup 1.0000x reference)
import functools

import jax
import jax.numpy as jnp
from jax import lax
from jax.experimental import pallas as pl
from jax.experimental.pallas import tpu as pltpu
from jax.experimental.pallas import tpu_sc as plsc

EPS = 1e-08


def _sc_gather(V, D, B, CH):
    info = plsc.get_sparse_core_info()
    NC, NS = info.num_cores, info.num_subcores
    NW = NC * NS
    b_per_w = B // NW
    assert b_per_w % CH == 0
    mesh = plsc.VectorSubcoreMesh(core_axis_name="c", subcore_axis_name="s")

    @functools.partial(
        pl.kernel,
        mesh=mesh,
        compiler_params=pltpu.CompilerParams(use_tc_tiling_on_sc=False),
        out_type=jax.ShapeDtypeStruct((B, D), jnp.float32),
        scratch_types=[
            pltpu.VMEM((CH,), jnp.int32),
            pltpu.VMEM((CH, D), jnp.float32),
            pltpu.SemaphoreType.DMA,
        ],
    )
    def k(table_hbm, idx_hbm, out_hbm, idx_v, rows_v, sem):
        wid = lax.axis_index("s") * NC + lax.axis_index("c")
        base = wid * b_per_w
        for c in range(b_per_w // CH):
            off = base + c * CH
            pltpu.sync_copy(idx_hbm.at[pl.ds(off, CH)], idx_v)
            pltpu.async_copy(table_hbm.at[idx_v], rows_v, sem).wait()
            pltpu.sync_copy(rows_v, out_hbm.at[pl.ds(off, CH)])

    return k


def _dice2(s, alpha):
    mean = jnp.mean(s, axis=0)
    var = jnp.mean((s - mean) ** 2, axis=0)
    p = jax.nn.sigmoid((s - mean) / jnp.sqrt(var + EPS))
    return p * s + (1.0 - p) * alpha * s


def _dice3(s, alpha):
    st = jnp.transpose(s, (0, 2, 1))
    mean = jnp.mean(st, axis=(0, 2), keepdims=True)
    var = jnp.mean((st - mean) ** 2, axis=(0, 2), keepdims=True)
    p = jax.nn.sigmoid((st - mean) / jnp.sqrt(var + EPS))
    out = p * st + (1.0 - p) * alpha * st
    return jnp.transpose(out, (0, 2, 1))


def kernel(user_id, beh_item, beh_cate, cand_item, cand_cate, ctx_id, mask,
           E_user, E_item, E_cate, E_ctx,
           W_a1, b_a1, alpha_a, W_a2, b_a2,
           W_f1, b_f1, alpha_f1, W_f2, b_f2, alpha_f2, W_f3, b_f3):
    B, L = beh_item.shape
    D = E_item.shape[1]
    bi = beh_item.reshape(-1).astype(jnp.int32)
    gi = _sc_gather(E_item.shape[0], D, B * L, 1600)(E_item, bi)

    user_embeds = jnp.take(E_user, user_id, axis=0)
    item_embeds = jnp.concatenate(
        [gi.reshape(B, L, D), jnp.take(E_cate, beh_cate, axis=0)], axis=-1)
    cand_embeds = jnp.concatenate(
        [jnp.take(E_item, cand_item, axis=0), jnp.take(E_cate, cand_cate, axis=0)],
        axis=-1)
    cand_exp = jnp.broadcast_to(cand_embeds[:, None, :],
                                (B, L, cand_embeds.shape[1]))
    ctx_embeds = jnp.take(E_ctx, ctx_id, axis=0)
    attn_inputs = jnp.concatenate(
        [item_embeds, cand_exp, cand_exp - item_embeds, cand_exp * item_embeds],
        axis=-1)
    h = attn_inputs @ W_a1 + b_a1
    h = _dice3(h, alpha_a)
    attn_weights = h @ W_a2 + b_a2
    attn_weights = attn_weights * mask[:, :, None]
    item_pooling = jnp.sum(attn_weights * item_embeds, axis=1)
    fc_in = jnp.concatenate(
        [user_embeds, item_pooling, cand_embeds, ctx_embeds], axis=-1)
    x = fc_in @ W_f1 + b_f1
    x = _dice2(x, alpha_f1)
    x = x @ W_f2 + b_f2
    x = _dice2(x, alpha_f2)
    x = x @ W_f3 + b_f3
    return jax.nn.softmax(x, axis=-1)


# speedup vs baseline: 2.6420x; 2.6420x over previous
import functools

import jax
import jax.numpy as jnp
from jax import lax
from jax.experimental import pallas as pl
from jax.experimental.pallas import tpu as pltpu
from jax.experimental.pallas import tpu_sc as plsc

EPS = 1e-08
F32 = jnp.float32


def _sc_gather_all(V_item, V_cate, V_user, V_ctx, NBIG, B, D):
    info = plsc.get_sparse_core_info()
    NC, NS = info.num_cores, info.num_subcores
    NW = NC * NS
    per_w = NBIG // NW
    CH = 1600
    NCH = per_w // CH
    SB = B // NW
    P = NBIG // 4
    mesh = plsc.VectorSubcoreMesh(core_axis_name="c", subcore_axis_name="s")

    @functools.partial(
        pl.kernel,
        mesh=mesh,
        compiler_params=pltpu.CompilerParams(use_tc_tiling_on_sc=False),
        out_type=(
            jax.ShapeDtypeStruct((NBIG, D), F32),
            jax.ShapeDtypeStruct((NBIG, D), F32),
            jax.ShapeDtypeStruct((B, D), F32),
            jax.ShapeDtypeStruct((B, D), F32),
            jax.ShapeDtypeStruct((B, D), F32),
            jax.ShapeDtypeStruct((B, D), F32),
        ),
        scratch_types=[
            pltpu.VMEM((CH,), jnp.int32),
            pltpu.VMEM((CH, D), F32),
            pltpu.VMEM((SB,), jnp.int32),
            pltpu.VMEM((SB, D), F32),
            pltpu.SemaphoreType.DMA,
        ],
    )
    def k(Ei, Ec, Eu, Ex, bi, bc, cand_i, cand_c, uid, ctxid,
          gi_out, gc_out, ci_out, cc_out, ui_out, xi_out,
          idx_v, rows_v, sidx_v, srows_v, sem):
        wid = lax.axis_index("s") * NC + lax.axis_index("c")
        for tab, idxarr, outarr in ((Ei, bi, gi_out), (Ec, bc, gc_out)):
            base = wid * per_w
            for c in range(NCH):
                off = base + c * CH
                pltpu.sync_copy(idxarr.at[pl.ds(off, CH)], idx_v)
                pltpu.async_copy(tab.at[idx_v], rows_v, sem).wait()
                pltpu.sync_copy(rows_v, outarr.at[pl.ds(off, CH)])
        for tab, idxarr, outarr in ((Ei, cand_i, ci_out), (Ec, cand_c, cc_out),
                                    (Eu, uid, ui_out), (Ex, ctxid, xi_out)):
            off = wid * SB
            pltpu.sync_copy(idxarr.at[pl.ds(off, SB)], sidx_v)
            pltpu.async_copy(tab.at[sidx_v], srows_v, sem).wait()
            pltpu.sync_copy(srows_v, outarr.at[pl.ds(off, SB)])

    return k


def _attn_kernel(NB, Bb, RB, N, ci_ref, cc_ref, gi_ref, gc_ref, mask_ref,
                 wb_ref, wc_ref, t36_ref, t32_ref, rep_ref, rept_ref,
                 wa2_ref, r4_ref, ai_ref, ac_ref, ba1_ref, alpha_ref, ba2_ref,
                 out_ref, ssum, ssq):
    i = pl.program_id(0)

    ci = ci_ref[...]
    cc = cc_ref[...]
    gi = gi_ref[...]
    gc = gc_ref[...]
    rep = rep_ref[...]
    wc = wc_ref[...]

    q = jnp.dot(ci, wc[0:32, :], preferred_element_type=F32)
    q = q + jnp.dot(cc, wc[32:64, :], preferred_element_type=F32)
    q144 = jnp.dot(q, t36_ref[...], preferred_element_type=F32)
    qrows = jnp.dot(rep, q144, preferred_element_type=F32)
    ci4 = jnp.dot(ci, t32_ref[...], preferred_element_type=F32)
    cc4 = jnp.dot(cc, t32_ref[...], preferred_element_type=F32)
    cirows = jnp.dot(rep, ci4, preferred_element_type=F32)
    ccrows = jnp.dot(rep, cc4, preferred_element_type=F32)
    prodi = gi * cirows
    prodc = gc * ccrows
    wb = wb_ref[...]
    h = jnp.dot(gi, wb[0:128, :], preferred_element_type=F32)
    h = h + jnp.dot(gc, wb[128:256, :], preferred_element_type=F32)
    h = h + jnp.dot(prodi, wb[256:384, :], preferred_element_type=F32)
    h = h + jnp.dot(prodc, wb[384:512, :], preferred_element_type=F32)
    h = h + qrows + ba1_ref[...]

    @pl.when(i == 0)
    def _init():
        ssum[...] = jnp.zeros_like(ssum)
        ssq[...] = jnp.zeros_like(ssq)

    @pl.when(i < NB)
    def _sweep1():
        ssum[...] += jnp.sum(h, axis=0, keepdims=True)
        ssq[...] += jnp.sum(h * h, axis=0, keepdims=True)
        out_ref[...] = jnp.zeros_like(out_ref)

    @pl.when(i >= NB)
    def _sweep2():
        t36 = t36_ref[...]
        s36 = jnp.dot(ssum[...], t36.T, preferred_element_type=F32)
        e36 = jnp.dot(ssq[...], t36.T, preferred_element_type=F32)
        m36 = s36 / N
        v36 = e36 / N - m36 * m36
        m144 = jnp.dot(m36, t36, preferred_element_type=F32)
        v144 = jnp.dot(v36, t36, preferred_element_type=F32)
        p = jax.nn.sigmoid((h - m144) * lax.rsqrt(v144 + EPS))
        d = h * (p + (1.0 - p) * alpha_ref[...])
        w4 = jnp.dot(d, wa2_ref[...], preferred_element_type=F32) + ba2_ref[...]
        w4 = w4 * mask_ref[...]
        wbig = jnp.dot(w4, r4_ref[...], preferred_element_type=F32)
        rept = rept_ref[...]
        si = jnp.dot(rept, wbig * gi, preferred_element_type=F32)
        sc2 = jnp.dot(rept, wbig * gc, preferred_element_type=F32)
        out_ref[...] = (jnp.dot(si, ai_ref[...], preferred_element_type=F32)
                        + jnp.dot(sc2, ac_ref[...], preferred_element_type=F32))


def _attention(gi, gc, ci, cc, mask_p, W_a1, b_a1, alpha_a, W_a2, b_a2,
               B, L):
    NB = 16
    Bb = B // NB
    RB = Bb * (L // 4)
    P = B * L // 4
    N = B * L
    eye4 = jnp.eye(4, dtype=F32)
    W1, W2, W3, W4 = (W_a1[0:64], W_a1[64:128], W_a1[128:192], W_a1[192:256])
    Wi = W1 - W3
    Wc = W2 + W3
    wb = jnp.concatenate([
        jnp.kron(eye4, Wi[0:32]), jnp.kron(eye4, Wi[32:64]),
        jnp.kron(eye4, W4[0:32]), jnp.kron(eye4, W4[32:64])], axis=0)
    t36 = jnp.tile(jnp.eye(36, dtype=F32), (1, 4))
    t32 = jnp.tile(jnp.eye(32, dtype=F32), (1, 4))
    rep = jnp.kron(jnp.eye(Bb, dtype=F32), jnp.ones((L // 4, 1), F32))
    rept = rep.T
    wa2 = jnp.kron(eye4, W_a2)
    r4 = jnp.kron(eye4, jnp.ones((1, 32), F32))
    t32v = jnp.tile(jnp.eye(32, dtype=F32), (4, 1))
    ai = jnp.concatenate([t32v, jnp.zeros((128, 96), F32)], axis=1)
    ac = jnp.concatenate([jnp.zeros((128, 32), F32), t32v,
                          jnp.zeros((128, 64), F32)], axis=1)
    ba1 = jnp.tile(b_a1.reshape(1, 36), (1, 4))
    alpha = jnp.tile(alpha_a.reshape(1, 36), (1, 4))
    ba2 = jnp.tile(b_a2.reshape(1, 1), (1, 4))

    def cmap(i):
        return (0, 0)

    def bmap(i):
        return (i % NB, 0)

    grid = (2 * NB,)
    out = pl.pallas_call(
        functools.partial(_attn_kernel, NB, Bb, RB, float(N)),
        grid=grid,
        in_specs=[
            pl.BlockSpec((Bb, 32), bmap),
            pl.BlockSpec((Bb, 32), bmap),
            pl.BlockSpec((RB, 128), bmap),
            pl.BlockSpec((RB, 128), bmap),
            pl.BlockSpec((RB, 4), bmap),
            pl.BlockSpec((512, 144), cmap),
            pl.BlockSpec((64, 36), cmap),
            pl.BlockSpec((36, 144), cmap),
            pl.BlockSpec((32, 128), cmap),
            pl.BlockSpec((RB, Bb), cmap),
            pl.BlockSpec((Bb, RB), cmap),
            pl.BlockSpec((144, 4), cmap),
            pl.BlockSpec((4, 128), cmap),
            pl.BlockSpec((128, 128), cmap),
            pl.BlockSpec((128, 128), cmap),
            pl.BlockSpec((1, 144), cmap),
            pl.BlockSpec((1, 144), cmap),
            pl.BlockSpec((1, 4), cmap),
        ],
        out_specs=pl.BlockSpec((Bb, 128), bmap),
        out_shape=jax.ShapeDtypeStruct((B, 128), F32),
        scratch_shapes=[
            pltpu.VMEM((1, 144), F32),
            pltpu.VMEM((1, 144), F32),
        ],
        compiler_params=pltpu.CompilerParams(
            dimension_semantics=("arbitrary",)),
    )(ci, cc, gi, gc, mask_p, wb, Wc, t36, t32, rep, rept, wa2, r4, ai, ac,
      ba1, alpha, ba2)
    return out


def _fc_kernel(Bf, ui_ref, pool_ref, ci_ref, cc_ref, xi_ref,
               w1_ref, b1_ref, a1_ref, w2_ref, b2_ref, a2_ref,
               w3_ref, b3_ref, out_ref):
    fc_in = jnp.concatenate(
        [ui_ref[...], pool_ref[...][:, 0:64], ci_ref[...], cc_ref[...],
         xi_ref[...]], axis=1)
    x = jnp.dot(fc_in, w1_ref[...], preferred_element_type=F32) + b1_ref[...]
    mean = jnp.mean(x, axis=0, keepdims=True)
    var = jnp.mean((x - mean) ** 2, axis=0, keepdims=True)
    p = jax.nn.sigmoid((x - mean) / jnp.sqrt(var + EPS))
    x = x * (p + (1.0 - p) * a1_ref[...])
    x = jnp.dot(x, w2_ref[...], preferred_element_type=F32) + b2_ref[...]
    mean = jnp.mean(x, axis=0, keepdims=True)
    var = jnp.mean((x - mean) ** 2, axis=0, keepdims=True)
    p = jax.nn.sigmoid((x - mean) / jnp.sqrt(var + EPS))
    x = x * (p + (1.0 - p) * a2_ref[...])
    x = jnp.dot(x, w3_ref[...], preferred_element_type=F32) + b3_ref[...]
    m = jnp.max(x, axis=-1, keepdims=True)
    e = jnp.exp(x - m)
    out_ref[...] = e / jnp.sum(e, axis=-1, keepdims=True)


def _fc_top(pooled, ui, ci, cc, xi, W_f1, b_f1, alpha_f1, W_f2, b_f2,
            alpha_f2, W_f3, b_f3, B):
    return pl.pallas_call(
        functools.partial(_fc_kernel, B),
        out_shape=jax.ShapeDtypeStruct((B, 2), F32),
    )(ui, pooled, ci, cc, xi,
      W_f1, b_f1.reshape(1, -1), alpha_f1.reshape(1, -1),
      W_f2, b_f2.reshape(1, -1), alpha_f2.reshape(1, -1),
      W_f3, b_f3.reshape(1, -1))


def kernel(user_id, beh_item, beh_cate, cand_item, cand_cate, ctx_id, mask,
           E_user, E_item, E_cate, E_ctx,
           W_a1, b_a1, alpha_a, W_a2, b_a2,
           W_f1, b_f1, alpha_f1, W_f2, b_f2, alpha_f2, W_f3, b_f3):
    B, L = beh_item.shape
    D = E_item.shape[1]
    N = B * L
    bi = beh_item.reshape(-1).astype(jnp.int32)
    bc = beh_cate.reshape(-1).astype(jnp.int32)
    gi, gc, ci, cc, ui, xi = _sc_gather_all(
        E_item.shape[0], E_cate.shape[0], E_user.shape[0], E_ctx.shape[0],
        N, B, D,
    )(E_item, E_cate, E_user, E_ctx, bi, bc,
      cand_item.astype(jnp.int32), cand_cate.astype(jnp.int32),
      user_id.astype(jnp.int32), ctx_id.astype(jnp.int32))
    gi = gi.reshape(N // 4, 128)
    gc = gc.reshape(N // 4, 128)

    mask_p = mask.reshape(N // 4, 4)
    pooled = _attention(gi, gc, ci, cc, mask_p, W_a1, b_a1, alpha_a,
                        W_a2, b_a2, B, L)
    return _fc_top(pooled, ui, ci, cc, xi, W_f1, b_f1, alpha_f1,
                   W_f2, b_f2, alpha_f2, W_f3, b_f3, B)


# speedup vs baseline: 2.6964x; 1.0206x over previous
import functools

import jax
import jax.numpy as jnp
from jax import lax
from jax.experimental import pallas as pl
from jax.experimental.pallas import tpu as pltpu
from jax.experimental.pallas import tpu_sc as plsc

EPS = 1e-08
F32 = jnp.float32


_TR_CHT = 2048
_TR_NI = 124
_TR_Q = _TR_CHT * _TR_NI


def _tr_kernel(in_ref, sel_ref, out_ref):
    j = pl.program_id(1)
    b = in_ref[...]
    t = lax.dot_general(b, jnp.eye(32, dtype=b.dtype),
                        (((0,), (0,)), ((), ())),
                        preferred_element_type=F32)
    contrib = jnp.dot(t, sel_ref[:, pl.ds(j * 128, 128)],
                      preferred_element_type=F32)

    @pl.when(j == 0)
    def _first():
        out_ref[...] = contrib

    @pl.when(j > 0)
    def _rest():
        out_ref[...] += contrib


def _row_majorize(tabT, V):
    eye32 = jnp.eye(32, dtype=F32)
    sels = jnp.concatenate(
        [jnp.pad(eye32, ((0, 0), (32 * j, 96 - 32 * j))) for j in range(4)],
        axis=1)
    grid = (_TR_NI, 4)
    nmax = (V - 1) // _TR_CHT
    return pl.pallas_call(
        _tr_kernel,
        grid=grid,
        in_specs=[
            pl.BlockSpec((32, _TR_CHT),
                         lambda i, j: (0, jnp.minimum(j * _TR_NI + i, nmax))),
            pl.BlockSpec((32, 512), lambda i, j: (0, 0)),
        ],
        out_specs=pl.BlockSpec((_TR_CHT, 128), lambda i, j: (i, 0)),
        out_shape=jax.ShapeDtypeStruct((_TR_Q, 128), F32),
        compiler_params=pltpu.CompilerParams(
            dimension_semantics=("arbitrary", "arbitrary")),
    )(tabT, sels)


def _sc_gather_all(V_item, V_cate, V_user, V_ctx, NBIG, B, D):
    info = plsc.get_sparse_core_info()
    NC, NS = info.num_cores, info.num_subcores
    NW = NC * NS
    per_w = NBIG // NW
    CH = 1600
    NCH = per_w // CH
    SB = B // NW
    P = NBIG // 4
    mesh = plsc.VectorSubcoreMesh(core_axis_name="c", subcore_axis_name="s")

    @functools.partial(
        pl.kernel,
        mesh=mesh,
        compiler_params=pltpu.CompilerParams(use_tc_tiling_on_sc=False),
        out_type=(
            jax.ShapeDtypeStruct((NBIG, D), F32),
            jax.ShapeDtypeStruct((NBIG, D), F32),
            jax.ShapeDtypeStruct((B, D), F32),
            jax.ShapeDtypeStruct((B, D), F32),
            jax.ShapeDtypeStruct((B, D), F32),
            jax.ShapeDtypeStruct((B, D), F32),
        ),
        scratch_types=[
            pltpu.VMEM((CH,), jnp.int32),
            pltpu.VMEM((CH, D), F32),
            pltpu.VMEM((SB,), jnp.int32),
            pltpu.VMEM((SB, D), F32),
            pltpu.SemaphoreType.DMA,
        ],
    )
    def k(Ei, Ec, Eu, Ex, bi, bc, cand_i, cand_c, uid, ctxid,
          gi_out, gc_out, ci_out, cc_out, ui_out, xi_out,
          idx_v, rows_v, sidx_v, srows_v, sem):
        wid = lax.axis_index("s") * NC + lax.axis_index("c")
        for tab, idxarr, outarr in ((Ei, bi, gi_out), (Ec, bc, gc_out)):
            base = wid * per_w
            for c in range(NCH):
                off = base + c * CH
                pltpu.sync_copy(idxarr.at[pl.ds(off, CH)], idx_v)
                pltpu.async_copy(tab.at[idx_v], rows_v, sem).wait()
                pltpu.sync_copy(rows_v, outarr.at[pl.ds(off, CH)])
        for tab, idxarr, outarr in ((Ei, cand_i, ci_out), (Ec, cand_c, cc_out),
                                    (Eu, uid, ui_out), (Ex, ctxid, xi_out)):
            off = wid * SB
            pltpu.sync_copy(idxarr.at[pl.ds(off, SB)], sidx_v)
            pltpu.async_copy(tab.at[sidx_v], srows_v, sem).wait()
            pltpu.sync_copy(srows_v, outarr.at[pl.ds(off, SB)])

    return k


def _attn_kernel(NB, Bb, RB, N, ci_ref, cc_ref, gi_ref, gc_ref, mask_ref,
                 wb_ref, wc_ref, t36_ref, t32_ref, rep_ref, rept_ref,
                 wa2_ref, r4_ref, ai_ref, ac_ref, ba1_ref, alpha_ref, ba2_ref,
                 out_ref, ssum, ssq):
    i = pl.program_id(0)

    ci = ci_ref[...]
    cc = cc_ref[...]
    gi = gi_ref[...]
    gc = gc_ref[...]
    rep = rep_ref[...]
    wc = wc_ref[...]

    q = jnp.dot(ci, wc[0:32, :], preferred_element_type=F32)
    q = q + jnp.dot(cc, wc[32:64, :], preferred_element_type=F32)
    q144 = jnp.dot(q, t36_ref[...], preferred_element_type=F32)
    qrows = jnp.dot(rep, q144, preferred_element_type=F32)
    ci4 = jnp.dot(ci, t32_ref[...], preferred_element_type=F32)
    cc4 = jnp.dot(cc, t32_ref[...], preferred_element_type=F32)
    cirows = jnp.dot(rep, ci4, preferred_element_type=F32)
    ccrows = jnp.dot(rep, cc4, preferred_element_type=F32)
    prodi = gi * cirows
    prodc = gc * ccrows
    wb = wb_ref[...]
    h = jnp.dot(gi, wb[0:128, :], preferred_element_type=F32)
    h = h + jnp.dot(gc, wb[128:256, :], preferred_element_type=F32)
    h = h + jnp.dot(prodi, wb[256:384, :], preferred_element_type=F32)
    h = h + jnp.dot(prodc, wb[384:512, :], preferred_element_type=F32)
    h = h + qrows + ba1_ref[...]

    @pl.when(i == 0)
    def _init():
        ssum[...] = jnp.zeros_like(ssum)
        ssq[...] = jnp.zeros_like(ssq)

    @pl.when(i < NB)
    def _sweep1():
        ssum[...] += jnp.sum(h, axis=0, keepdims=True)
        ssq[...] += jnp.sum(h * h, axis=0, keepdims=True)
        out_ref[...] = jnp.zeros_like(out_ref)

    @pl.when(i >= NB)
    def _sweep2():
        t36 = t36_ref[...]
        s36 = jnp.dot(ssum[...], t36.T, preferred_element_type=F32)
        e36 = jnp.dot(ssq[...], t36.T, preferred_element_type=F32)
        m36 = s36 / N
        v36 = e36 / N - m36 * m36
        m144 = jnp.dot(m36, t36, preferred_element_type=F32)
        v144 = jnp.dot(v36, t36, preferred_element_type=F32)
        p = jax.nn.sigmoid((h - m144) * lax.rsqrt(v144 + EPS))
        d = h * (p + (1.0 - p) * alpha_ref[...])
        w4 = jnp.dot(d, wa2_ref[...], preferred_element_type=F32) + ba2_ref[...]
        w4 = w4 * mask_ref[...]
        wbig = jnp.dot(w4, r4_ref[...], preferred_element_type=F32)
        rept = rept_ref[...]
        si = jnp.dot(rept, wbig * gi, preferred_element_type=F32)
        sc2 = jnp.dot(rept, wbig * gc, preferred_element_type=F32)
        out_ref[...] = (jnp.dot(si, ai_ref[...], preferred_element_type=F32)
                        + jnp.dot(sc2, ac_ref[...], preferred_element_type=F32))


def _attention(gi, gc, ci, cc, mask_p, W_a1, b_a1, alpha_a, W_a2, b_a2,
               B, L):
    NB = 16
    Bb = B // NB
    RB = Bb * (L // 4)
    P = B * L // 4
    N = B * L
    eye4 = jnp.eye(4, dtype=F32)
    W1, W2, W3, W4 = (W_a1[0:64], W_a1[64:128], W_a1[128:192], W_a1[192:256])
    Wi = W1 - W3
    Wc = W2 + W3
    wb = jnp.concatenate([
        jnp.kron(eye4, Wi[0:32]), jnp.kron(eye4, Wi[32:64]),
        jnp.kron(eye4, W4[0:32]), jnp.kron(eye4, W4[32:64])], axis=0)
    t36 = jnp.tile(jnp.eye(36, dtype=F32), (1, 4))
    t32 = jnp.tile(jnp.eye(32, dtype=F32), (1, 4))
    rep = jnp.kron(jnp.eye(Bb, dtype=F32), jnp.ones((L // 4, 1), F32))
    rept = rep.T
    wa2 = jnp.kron(eye4, W_a2)
    r4 = jnp.kron(eye4, jnp.ones((1, 32), F32))
    t32v = jnp.tile(jnp.eye(32, dtype=F32), (4, 1))
    ai = jnp.concatenate([t32v, jnp.zeros((128, 96), F32)], axis=1)
    ac = jnp.concatenate([jnp.zeros((128, 32), F32), t32v,
                          jnp.zeros((128, 64), F32)], axis=1)
    ba1 = jnp.tile(b_a1.reshape(1, 36), (1, 4))
    alpha = jnp.tile(alpha_a.reshape(1, 36), (1, 4))
    ba2 = jnp.tile(b_a2.reshape(1, 1), (1, 4))

    def cmap(i):
        return (0, 0)

    def bmap(i):
        return (i % NB, 0)

    grid = (2 * NB,)
    out = pl.pallas_call(
        functools.partial(_attn_kernel, NB, Bb, RB, float(N)),
        grid=grid,
        in_specs=[
            pl.BlockSpec((Bb, 32), bmap),
            pl.BlockSpec((Bb, 32), bmap),
            pl.BlockSpec((RB, 128), bmap),
            pl.BlockSpec((RB, 128), bmap),
            pl.BlockSpec((RB, 4), bmap),
            pl.BlockSpec((512, 144), cmap),
            pl.BlockSpec((64, 36), cmap),
            pl.BlockSpec((36, 144), cmap),
            pl.BlockSpec((32, 128), cmap),
            pl.BlockSpec((RB, Bb), cmap),
            pl.BlockSpec((Bb, RB), cmap),
            pl.BlockSpec((144, 4), cmap),
            pl.BlockSpec((4, 128), cmap),
            pl.BlockSpec((128, 128), cmap),
            pl.BlockSpec((128, 128), cmap),
            pl.BlockSpec((1, 144), cmap),
            pl.BlockSpec((1, 144), cmap),
            pl.BlockSpec((1, 4), cmap),
        ],
        out_specs=pl.BlockSpec((Bb, 128), bmap),
        out_shape=jax.ShapeDtypeStruct((B, 128), F32),
        scratch_shapes=[
            pltpu.VMEM((1, 144), F32),
            pltpu.VMEM((1, 144), F32),
        ],
        compiler_params=pltpu.CompilerParams(
            dimension_semantics=("arbitrary",)),
    )(ci, cc, gi, gc, mask_p, wb, Wc, t36, t32, rep, rept, wa2, r4, ai, ac,
      ba1, alpha, ba2)
    return out


def _fc_kernel(Bf, ui_ref, pool_ref, ci_ref, cc_ref, xi_ref,
               w1_ref, b1_ref, a1_ref, w2_ref, b2_ref, a2_ref,
               w3_ref, b3_ref, out_ref):
    fc_in = jnp.concatenate(
        [ui_ref[...], pool_ref[...][:, 0:64], ci_ref[...], cc_ref[...],
         xi_ref[...]], axis=1)
    x = jnp.dot(fc_in, w1_ref[...], preferred_element_type=F32) + b1_ref[...]
    mean = jnp.mean(x, axis=0, keepdims=True)
    var = jnp.mean((x - mean) ** 2, axis=0, keepdims=True)
    p = jax.nn.sigmoid((x - mean) / jnp.sqrt(var + EPS))
    x = x * (p + (1.0 - p) * a1_ref[...])
    x = jnp.dot(x, w2_ref[...], preferred_element_type=F32) + b2_ref[...]
    mean = jnp.mean(x, axis=0, keepdims=True)
    var = jnp.mean((x - mean) ** 2, axis=0, keepdims=True)
    p = jax.nn.sigmoid((x - mean) / jnp.sqrt(var + EPS))
    x = x * (p + (1.0 - p) * a2_ref[...])
    x = jnp.dot(x, w3_ref[...], preferred_element_type=F32) + b3_ref[...]
    m = jnp.max(x, axis=-1, keepdims=True)
    e = jnp.exp(x - m)
    out_ref[...] = e / jnp.sum(e, axis=-1, keepdims=True)


def _fc_top(pooled, ui, ci, cc, xi, W_f1, b_f1, alpha_f1, W_f2, b_f2,
            alpha_f2, W_f3, b_f3, B):
    return pl.pallas_call(
        functools.partial(_fc_kernel, B),
        out_shape=jax.ShapeDtypeStruct((B, 2), F32),
    )(ui, pooled, ci, cc, xi,
      W_f1, b_f1.reshape(1, -1), alpha_f1.reshape(1, -1),
      W_f2, b_f2.reshape(1, -1), alpha_f2.reshape(1, -1),
      W_f3, b_f3.reshape(1, -1))


def kernel(user_id, beh_item, beh_cate, cand_item, cand_cate, ctx_id, mask,
           E_user, E_item, E_cate, E_ctx,
           W_a1, b_a1, alpha_a, W_a2, b_a2,
           W_f1, b_f1, alpha_f1, W_f2, b_f2, alpha_f2, W_f3, b_f3):
    B, L = beh_item.shape
    D = E_item.shape[1]
    N = B * L
    bi = beh_item.reshape(-1).astype(jnp.int32)
    bc = beh_cate.reshape(-1).astype(jnp.int32)
    V = E_item.shape[0]
    E_item_rm = _row_majorize(E_item.T, V).reshape(4 * _TR_Q, D)
    bi = (bi % _TR_Q) * 4 + bi // _TR_Q
    ct = cand_item.astype(jnp.int32)
    ct = (ct % _TR_Q) * 4 + ct // _TR_Q
    gi, gc, ci, cc, ui, xi = _sc_gather_all(
        4 * _TR_Q, E_cate.shape[0], E_user.shape[0], E_ctx.shape[0],
        N, B, D,
    )(E_item_rm, E_cate, E_user, E_ctx, bi, bc,
      ct, cand_cate.astype(jnp.int32),
      user_id.astype(jnp.int32), ctx_id.astype(jnp.int32))
    gi = gi.reshape(N // 4, 128)
    gc = gc.reshape(N // 4, 128)

    mask_p = mask.reshape(N // 4, 4)
    pooled = _attention(gi, gc, ci, cc, mask_p, W_a1, b_a1, alpha_a,
                        W_a2, b_a2, B, L)
    return _fc_top(pooled, ui, ci, cc, xi, W_f1, b_f1, alpha_f1,
                   W_f2, b_f2, alpha_f2, W_f3, b_f3, B)


# speedup vs baseline: 5.2025x; 1.9294x over previous
import functools

import jax
import jax.numpy as jnp
from jax import lax
from jax.experimental import pallas as pl
from jax.experimental.pallas import tpu as pltpu
from jax.experimental.pallas import tpu_sc as plsc

EPS = 1e-08
F32 = jnp.float32


_TR_CHT = 8192
_TR_NI = 31
_TR_Q = _TR_CHT * _TR_NI


def _tr_kernel(b0, b1, b2, b3, eye_ref, out_ref):
    b = jnp.concatenate([b0[...], b1[...], b2[...], b3[...]], axis=0)
    out_ref[...] = lax.dot_general(b, eye_ref[...],
                                   (((0,), (0,)), ((), ())),
                                   preferred_element_type=F32)


def _row_majorize(tabT, V):
    nmax = (V - 1) // _TR_CHT

    def make_map(j):
        return lambda i: (0, jnp.minimum(j * _TR_NI + i, nmax))

    return pl.pallas_call(
        _tr_kernel,
        grid=(_TR_NI,),
        in_specs=[pl.BlockSpec((32, _TR_CHT), make_map(j)) for j in range(4)]
        + [pl.BlockSpec((128, 128), lambda i: (0, 0))],
        out_specs=pl.BlockSpec((_TR_CHT, 128), lambda i: (i, 0)),
        out_shape=jax.ShapeDtypeStruct((_TR_Q, 128), F32),
    )(tabT, tabT, tabT, tabT, jnp.eye(128, dtype=F32))


def _sc_gather_all(V_item, V_cate, V_user, V_ctx, NBIG, B, D):
    info = plsc.get_sparse_core_info()
    NC, NS = info.num_cores, info.num_subcores
    NW = NC * NS
    per_w = NBIG // NW
    CH = 1600
    NCH = per_w // CH
    SB = B // NW
    P = NBIG // 4
    mesh = plsc.VectorSubcoreMesh(core_axis_name="c", subcore_axis_name="s")

    @functools.partial(
        pl.kernel,
        mesh=mesh,
        compiler_params=pltpu.CompilerParams(use_tc_tiling_on_sc=False),
        out_type=(
            jax.ShapeDtypeStruct((NBIG, D), F32),
            jax.ShapeDtypeStruct((NBIG, D), F32),
            jax.ShapeDtypeStruct((B, D), F32),
            jax.ShapeDtypeStruct((B, D), F32),
            jax.ShapeDtypeStruct((B, D), F32),
            jax.ShapeDtypeStruct((B, D), F32),
        ),
        scratch_types=[
            pltpu.VMEM((CH,), jnp.int32),
            pltpu.VMEM((CH, D), F32),
            pltpu.VMEM((SB,), jnp.int32),
            pltpu.VMEM((SB, D), F32),
            pltpu.SemaphoreType.DMA,
        ],
    )
    def k(Ei, Ec, Eu, Ex, bi, bc, cand_i, cand_c, uid, ctxid,
          gi_out, gc_out, ci_out, cc_out, ui_out, xi_out,
          idx_v, rows_v, sidx_v, srows_v, sem):
        wid = lax.axis_index("s") * NC + lax.axis_index("c")
        for tab, idxarr, outarr in ((Ei, bi, gi_out), (Ec, bc, gc_out)):
            base = wid * per_w
            for c in range(NCH):
                off = base + c * CH
                pltpu.sync_copy(idxarr.at[pl.ds(off, CH)], idx_v)
                pltpu.async_copy(tab.at[idx_v], rows_v, sem).wait()
                pltpu.sync_copy(rows_v, outarr.at[pl.ds(off, CH)])
        for tab, idxarr, outarr in ((Ei, cand_i, ci_out), (Ec, cand_c, cc_out),
                                    (Eu, uid, ui_out), (Ex, ctxid, xi_out)):
            off = wid * SB
            pltpu.sync_copy(idxarr.at[pl.ds(off, SB)], sidx_v)
            pltpu.async_copy(tab.at[sidx_v], srows_v, sem).wait()
            pltpu.sync_copy(srows_v, outarr.at[pl.ds(off, SB)])

    return k


def _attn_kernel(NB, Bb, RB, N, ci_ref, cc_ref, gi_ref, gc_ref, mask_ref,
                 wb_ref, wc_ref, t36_ref, t32_ref, rep_ref, rept_ref,
                 wa2_ref, r4_ref, ai_ref, ac_ref, ba1_ref, alpha_ref, ba2_ref,
                 out_ref, ssum, ssq):
    i = pl.program_id(0)

    ci = ci_ref[...]
    cc = cc_ref[...]
    gi = gi_ref[...]
    gc = gc_ref[...]
    rep = rep_ref[...]
    wc = wc_ref[...]

    q = jnp.dot(ci, wc[0:32, :], preferred_element_type=F32)
    q = q + jnp.dot(cc, wc[32:64, :], preferred_element_type=F32)
    q144 = jnp.dot(q, t36_ref[...], preferred_element_type=F32)
    qrows = jnp.dot(rep, q144, preferred_element_type=F32)
    ci4 = jnp.dot(ci, t32_ref[...], preferred_element_type=F32)
    cc4 = jnp.dot(cc, t32_ref[...], preferred_element_type=F32)
    cirows = jnp.dot(rep, ci4, preferred_element_type=F32)
    ccrows = jnp.dot(rep, cc4, preferred_element_type=F32)
    prodi = gi * cirows
    prodc = gc * ccrows
    wb = wb_ref[...]
    BF = jnp.bfloat16
    h = jnp.dot(gi.astype(BF), wb[0:128, :].astype(BF),
                preferred_element_type=F32)
    h = h + jnp.dot(gc.astype(BF), wb[128:256, :].astype(BF),
                    preferred_element_type=F32)
    h = h + jnp.dot(prodi.astype(BF), wb[256:384, :].astype(BF),
                    preferred_element_type=F32)
    h = h + jnp.dot(prodc.astype(BF), wb[384:512, :].astype(BF),
                    preferred_element_type=F32)
    h = h + qrows + ba1_ref[...]

    @pl.when(i == 0)
    def _init():
        ssum[...] = jnp.zeros_like(ssum)
        ssq[...] = jnp.zeros_like(ssq)

    @pl.when(i < NB)
    def _sweep1():
        ssum[...] += jnp.sum(h, axis=0, keepdims=True)
        ssq[...] += jnp.sum(h * h, axis=0, keepdims=True)
        out_ref[...] = jnp.zeros_like(out_ref)

    @pl.when(i >= NB)
    def _sweep2():
        t36 = t36_ref[...]
        s36 = jnp.dot(ssum[...], t36.T, preferred_element_type=F32)
        e36 = jnp.dot(ssq[...], t36.T, preferred_element_type=F32)
        m36 = s36 / N
        v36 = e36 / N - m36 * m36
        m144 = jnp.dot(m36, t36, preferred_element_type=F32)
        v144 = jnp.dot(v36, t36, preferred_element_type=F32)
        p = jax.nn.sigmoid((h - m144) * lax.rsqrt(v144 + EPS))
        d = h * (p + (1.0 - p) * alpha_ref[...])
        w4 = jnp.dot(d, wa2_ref[...], preferred_element_type=F32) + ba2_ref[...]
        w4 = w4 * mask_ref[...]
        wbig = jnp.dot(w4, r4_ref[...], preferred_element_type=F32)
        rept = rept_ref[...]
        si = jnp.dot(rept, wbig * gi, preferred_element_type=F32)
        sc2 = jnp.dot(rept, wbig * gc, preferred_element_type=F32)
        out_ref[...] = (jnp.dot(si, ai_ref[...], preferred_element_type=F32)
                        + jnp.dot(sc2, ac_ref[...], preferred_element_type=F32))


def _attention(gi, gc, ci, cc, mask_p, W_a1, b_a1, alpha_a, W_a2, b_a2,
               B, L):
    NB = 16
    Bb = B // NB
    RB = Bb * (L // 4)
    P = B * L // 4
    N = B * L
    eye4 = jnp.eye(4, dtype=F32)
    W1, W2, W3, W4 = (W_a1[0:64], W_a1[64:128], W_a1[128:192], W_a1[192:256])
    Wi = W1 - W3
    Wc = W2 + W3
    wb = jnp.concatenate([
        jnp.kron(eye4, Wi[0:32]), jnp.kron(eye4, Wi[32:64]),
        jnp.kron(eye4, W4[0:32]), jnp.kron(eye4, W4[32:64])], axis=0)
    t36 = jnp.tile(jnp.eye(36, dtype=F32), (1, 4))
    t32 = jnp.tile(jnp.eye(32, dtype=F32), (1, 4))
    rep = jnp.kron(jnp.eye(Bb, dtype=F32), jnp.ones((L // 4, 1), F32))
    rept = rep.T
    wa2 = jnp.kron(eye4, W_a2)
    r4 = jnp.kron(eye4, jnp.ones((1, 32), F32))
    t32v = jnp.tile(jnp.eye(32, dtype=F32), (4, 1))
    ai = jnp.concatenate([t32v, jnp.zeros((128, 96), F32)], axis=1)
    ac = jnp.concatenate([jnp.zeros((128, 32), F32), t32v,
                          jnp.zeros((128, 64), F32)], axis=1)
    ba1 = jnp.tile(b_a1.reshape(1, 36), (1, 4))
    alpha = jnp.tile(alpha_a.reshape(1, 36), (1, 4))
    ba2 = jnp.tile(b_a2.reshape(1, 1), (1, 4))

    def cmap(i):
        return (0, 0)

    def bmap(i):
        return (i % NB, 0)

    grid = (2 * NB,)
    out = pl.pallas_call(
        functools.partial(_attn_kernel, NB, Bb, RB, float(N)),
        grid=grid,
        in_specs=[
            pl.BlockSpec((Bb, 32), bmap),
            pl.BlockSpec((Bb, 32), bmap),
            pl.BlockSpec((RB, 128), bmap),
            pl.BlockSpec((RB, 128), bmap),
            pl.BlockSpec((RB, 4), bmap),
            pl.BlockSpec((512, 144), cmap),
            pl.BlockSpec((64, 36), cmap),
            pl.BlockSpec((36, 144), cmap),
            pl.BlockSpec((32, 128), cmap),
            pl.BlockSpec((RB, Bb), cmap),
            pl.BlockSpec((Bb, RB), cmap),
            pl.BlockSpec((144, 4), cmap),
            pl.BlockSpec((4, 128), cmap),
            pl.BlockSpec((128, 128), cmap),
            pl.BlockSpec((128, 128), cmap),
            pl.BlockSpec((1, 144), cmap),
            pl.BlockSpec((1, 144), cmap),
            pl.BlockSpec((1, 4), cmap),
        ],
        out_specs=pl.BlockSpec((Bb, 128), bmap),
        out_shape=jax.ShapeDtypeStruct((B, 128), F32),
        scratch_shapes=[
            pltpu.VMEM((1, 144), F32),
            pltpu.VMEM((1, 144), F32),
        ],
        compiler_params=pltpu.CompilerParams(
            dimension_semantics=("arbitrary",)),
    )(ci, cc, gi, gc, mask_p, wb, Wc, t36, t32, rep, rept, wa2, r4, ai, ac,
      ba1, alpha, ba2)
    return out


def _fc_kernel(Bf, ui_ref, pool_ref, ci_ref, cc_ref, xi_ref,
               w1_ref, b1_ref, a1_ref, w2_ref, b2_ref, a2_ref,
               w3_ref, b3_ref, out_ref):
    fc_in = jnp.concatenate(
        [ui_ref[...], pool_ref[...][:, 0:64], ci_ref[...], cc_ref[...],
         xi_ref[...]], axis=1)
    x = jnp.dot(fc_in, w1_ref[...], preferred_element_type=F32) + b1_ref[...]
    mean = jnp.mean(x, axis=0, keepdims=True)
    var = jnp.mean((x - mean) ** 2, axis=0, keepdims=True)
    p = jax.nn.sigmoid((x - mean) / jnp.sqrt(var + EPS))
    x = x * (p + (1.0 - p) * a1_ref[...])
    x = jnp.dot(x, w2_ref[...], preferred_element_type=F32) + b2_ref[...]
    mean = jnp.mean(x, axis=0, keepdims=True)
    var = jnp.mean((x - mean) ** 2, axis=0, keepdims=True)
    p = jax.nn.sigmoid((x - mean) / jnp.sqrt(var + EPS))
    x = x * (p + (1.0 - p) * a2_ref[...])
    x = jnp.dot(x, w3_ref[...], preferred_element_type=F32) + b3_ref[...]
    m = jnp.max(x, axis=-1, keepdims=True)
    e = jnp.exp(x - m)
    out_ref[...] = e / jnp.sum(e, axis=-1, keepdims=True)


def _fc_top(pooled, ui, ci, cc, xi, W_f1, b_f1, alpha_f1, W_f2, b_f2,
            alpha_f2, W_f3, b_f3, B):
    return pl.pallas_call(
        functools.partial(_fc_kernel, B),
        out_shape=jax.ShapeDtypeStruct((B, 2), F32),
    )(ui, pooled, ci, cc, xi,
      W_f1, b_f1.reshape(1, -1), alpha_f1.reshape(1, -1),
      W_f2, b_f2.reshape(1, -1), alpha_f2.reshape(1, -1),
      W_f3, b_f3.reshape(1, -1))


def kernel(user_id, beh_item, beh_cate, cand_item, cand_cate, ctx_id, mask,
           E_user, E_item, E_cate, E_ctx,
           W_a1, b_a1, alpha_a, W_a2, b_a2,
           W_f1, b_f1, alpha_f1, W_f2, b_f2, alpha_f2, W_f3, b_f3):
    B, L = beh_item.shape
    D = E_item.shape[1]
    N = B * L
    bi = beh_item.reshape(-1).astype(jnp.int32)
    bc = beh_cate.reshape(-1).astype(jnp.int32)
    V = E_item.shape[0]
    E_item_rm = _row_majorize(E_item.T, V).reshape(4 * _TR_Q, D)
    bi = (bi % _TR_Q) * 4 + bi // _TR_Q
    ct = cand_item.astype(jnp.int32)
    ct = (ct % _TR_Q) * 4 + ct // _TR_Q
    gi, gc, ci, cc, ui, xi = _sc_gather_all(
        4 * _TR_Q, E_cate.shape[0], E_user.shape[0], E_ctx.shape[0],
        N, B, D,
    )(E_item_rm, E_cate, E_user, E_ctx, bi, bc,
      ct, cand_cate.astype(jnp.int32),
      user_id.astype(jnp.int32), ctx_id.astype(jnp.int32))
    gi = gi.reshape(N // 4, 128)
    gc = gc.reshape(N // 4, 128)

    mask_p = mask.reshape(N // 4, 4)
    pooled = _attention(gi, gc, ci, cc, mask_p, W_a1, b_a1, alpha_a,
                        W_a2, b_a2, B, L)
    return _fc_top(pooled, ui, ci, cc, xi, W_f1, b_f1, alpha_f1,
                   W_f2, b_f2, alpha_f2, W_f3, b_f3, B)


# speedup vs baseline: 5.8924x; 1.1326x over previous
import functools

import jax
import jax.numpy as jnp
from jax import lax
from jax.experimental import pallas as pl
from jax.experimental.pallas import tpu as pltpu
from jax.experimental.pallas import tpu_sc as plsc

EPS = 1e-08
F32 = jnp.float32


_TR_CHT = 8192


def _tr_kernel(b0, b1, b2, b3, eye_ref, out_ref):
    b = jnp.concatenate([b0[...], b1[...], b2[...], b3[...]], axis=0)
    out_ref[...] = lax.dot_general(b, eye_ref[...],
                                   (((0,), (0,)), ((), ())),
                                   preferred_element_type=F32)


def _row_majorize(tabT, V):
    nmax = (V - 1) // _TR_CHT
    ni = -(-(V // 4) // _TR_CHT)
    q2 = ni * _TR_CHT

    def make_map(j):
        return lambda i: (0, jnp.minimum(j * ni + i, nmax))

    out = pl.pallas_call(
        _tr_kernel,
        grid=(ni,),
        in_specs=[pl.BlockSpec((32, _TR_CHT), make_map(j)) for j in range(4)]
        + [pl.BlockSpec((128, 128), lambda i: (0, 0))],
        out_specs=pl.BlockSpec((_TR_CHT, 128), lambda i: (i, 0)),
        out_shape=jax.ShapeDtypeStruct((q2, 128), F32),
    )(tabT, tabT, tabT, tabT, jnp.eye(128, dtype=F32))
    return out, q2


def _sc_gather_all(V_item, V_cate, V_user, V_ctx, NBIG, B, D):
    info = plsc.get_sparse_core_info()
    NC, NS = info.num_cores, info.num_subcores
    NW = NC * NS
    per_w = NBIG // NW
    CH = 1600
    NCH = per_w // CH
    SB = B // NW
    P = NBIG // 4
    mesh = plsc.VectorSubcoreMesh(core_axis_name="c", subcore_axis_name="s")

    @functools.partial(
        pl.kernel,
        mesh=mesh,
        compiler_params=pltpu.CompilerParams(use_tc_tiling_on_sc=False),
        out_type=(
            jax.ShapeDtypeStruct((NBIG, D), F32),
            jax.ShapeDtypeStruct((NBIG, D), F32),
            jax.ShapeDtypeStruct((B, D), F32),
            jax.ShapeDtypeStruct((B, D), F32),
            jax.ShapeDtypeStruct((B, D), F32),
            jax.ShapeDtypeStruct((B, D), F32),
        ),
        scratch_types=[
            pltpu.VMEM((CH,), jnp.int32),
            pltpu.VMEM((CH, D), F32),
            pltpu.VMEM((SB,), jnp.int32),
            pltpu.VMEM((SB, D), F32),
            pltpu.SemaphoreType.DMA,
        ],
    )
    def k(Ei, Ec, Eu, Ex, bi, bc, cand_i, cand_c, uid, ctxid,
          gi_out, gc_out, ci_out, cc_out, ui_out, xi_out,
          idx_v, rows_v, sidx_v, srows_v, sem):
        wid = lax.axis_index("s") * NC + lax.axis_index("c")
        for tab, idxarr, outarr in ((Ei, bi, gi_out), (Ec, bc, gc_out)):
            base = wid * per_w
            for c in range(NCH):
                off = base + c * CH
                pltpu.sync_copy(idxarr.at[pl.ds(off, CH)], idx_v)
                pltpu.async_copy(tab.at[idx_v], rows_v, sem).wait()
                pltpu.sync_copy(rows_v, outarr.at[pl.ds(off, CH)])
        for tab, idxarr, outarr in ((Ei, cand_i, ci_out), (Ec, cand_c, cc_out),
                                    (Eu, uid, ui_out), (Ex, ctxid, xi_out)):
            off = wid * SB
            pltpu.sync_copy(idxarr.at[pl.ds(off, SB)], sidx_v)
            pltpu.async_copy(tab.at[sidx_v], srows_v, sem).wait()
            pltpu.sync_copy(srows_v, outarr.at[pl.ds(off, SB)])

    return k


def _attn_kernel(NB, Bb, RB, N, ci_ref, cc_ref, gi_ref, gc_ref, mask_ref,
                 wb_ref, wc_ref, t36_ref, t32_ref, rep_ref, rept_ref,
                 wa2_ref, r4_ref, ai_ref, ac_ref, ba1_ref, alpha_ref, ba2_ref,
                 out_ref, ssum, ssq):
    i = pl.program_id(0)

    ci = ci_ref[...]
    cc = cc_ref[...]
    gi = gi_ref[...]
    gc = gc_ref[...]
    rep = rep_ref[...]
    wc = wc_ref[...]

    q = jnp.dot(ci, wc[0:32, :], preferred_element_type=F32)
    q = q + jnp.dot(cc, wc[32:64, :], preferred_element_type=F32)
    q144 = jnp.dot(q, t36_ref[...], preferred_element_type=F32)
    qrows = jnp.dot(rep, q144, preferred_element_type=F32)
    ci4 = jnp.dot(ci, t32_ref[...], preferred_element_type=F32)
    cc4 = jnp.dot(cc, t32_ref[...], preferred_element_type=F32)
    cirows = jnp.dot(rep, ci4, preferred_element_type=F32)
    ccrows = jnp.dot(rep, cc4, preferred_element_type=F32)
    prodi = gi * cirows
    prodc = gc * ccrows
    BF = jnp.bfloat16
    x = jnp.concatenate(
        [gi.astype(BF), gc.astype(BF), prodi.astype(BF), prodc.astype(BF)],
        axis=1)
    h = jnp.dot(x, wb_ref[...].astype(BF), preferred_element_type=F32)
    h = h + qrows + ba1_ref[...]

    @pl.when(i == 0)
    def _init():
        ssum[...] = jnp.zeros_like(ssum)
        ssq[...] = jnp.zeros_like(ssq)

    @pl.when(i < NB)
    def _sweep1():
        ssum[...] += jnp.sum(h, axis=0, keepdims=True)
        ssq[...] += jnp.sum(h * h, axis=0, keepdims=True)
        out_ref[...] = jnp.zeros_like(out_ref)

    @pl.when(i >= NB)
    def _sweep2():
        t36 = t36_ref[...]
        s36 = jnp.dot(ssum[...], t36.T, preferred_element_type=F32)
        e36 = jnp.dot(ssq[...], t36.T, preferred_element_type=F32)
        m36 = s36 / N
        v36 = e36 / N - m36 * m36
        m144 = jnp.dot(m36, t36, preferred_element_type=F32)
        v144 = jnp.dot(v36, t36, preferred_element_type=F32)
        p = jax.nn.sigmoid((h - m144) * lax.rsqrt(v144 + EPS))
        d = h * (p + (1.0 - p) * alpha_ref[...])
        w4 = jnp.dot(d, wa2_ref[...], preferred_element_type=F32) + ba2_ref[...]
        w4 = w4 * mask_ref[...]
        wbig = jnp.dot(w4, r4_ref[...], preferred_element_type=F32)
        rept = rept_ref[...]
        si = jnp.dot(rept, wbig * gi, preferred_element_type=F32)
        sc2 = jnp.dot(rept, wbig * gc, preferred_element_type=F32)
        out_ref[...] = (jnp.dot(si, ai_ref[...], preferred_element_type=F32)
                        + jnp.dot(sc2, ac_ref[...], preferred_element_type=F32))


def _attention(gi, gc, ci, cc, mask_p, W_a1, b_a1, alpha_a, W_a2, b_a2,
               B, L):
    NB = 16
    Bb = B // NB
    RB = Bb * (L // 4)
    P = B * L // 4
    N = B * L
    eye4 = jnp.eye(4, dtype=F32)
    W1, W2, W3, W4 = (W_a1[0:64], W_a1[64:128], W_a1[128:192], W_a1[192:256])
    Wi = W1 - W3
    Wc = W2 + W3
    wb = jnp.concatenate([
        jnp.kron(eye4, Wi[0:32]), jnp.kron(eye4, Wi[32:64]),
        jnp.kron(eye4, W4[0:32]), jnp.kron(eye4, W4[32:64])], axis=0)
    t36 = jnp.tile(jnp.eye(36, dtype=F32), (1, 4))
    t32 = jnp.tile(jnp.eye(32, dtype=F32), (1, 4))
    rep = jnp.kron(jnp.eye(Bb, dtype=F32), jnp.ones((L // 4, 1), F32))
    rept = rep.T
    wa2 = jnp.kron(eye4, W_a2)
    r4 = jnp.kron(eye4, jnp.ones((1, 32), F32))
    t32v = jnp.tile(jnp.eye(32, dtype=F32), (4, 1))
    ai = jnp.concatenate([t32v, jnp.zeros((128, 96), F32)], axis=1)
    ac = jnp.concatenate([jnp.zeros((128, 32), F32), t32v,
                          jnp.zeros((128, 64), F32)], axis=1)
    ba1 = jnp.tile(b_a1.reshape(1, 36), (1, 4))
    alpha = jnp.tile(alpha_a.reshape(1, 36), (1, 4))
    ba2 = jnp.tile(b_a2.reshape(1, 1), (1, 4))

    def cmap(i):
        return (0, 0)

    def bmap(i):
        return (i % NB, 0)

    grid = (2 * NB,)
    out = pl.pallas_call(
        functools.partial(_attn_kernel, NB, Bb, RB, float(N)),
        grid=grid,
        in_specs=[
            pl.BlockSpec((Bb, 32), bmap),
            pl.BlockSpec((Bb, 32), bmap),
            pl.BlockSpec((RB, 128), bmap),
            pl.BlockSpec((RB, 128), bmap),
            pl.BlockSpec((RB, 4), bmap),
            pl.BlockSpec((512, 144), cmap),
            pl.BlockSpec((64, 36), cmap),
            pl.BlockSpec((36, 144), cmap),
            pl.BlockSpec((32, 128), cmap),
            pl.BlockSpec((RB, Bb), cmap),
            pl.BlockSpec((Bb, RB), cmap),
            pl.BlockSpec((144, 4), cmap),
            pl.BlockSpec((4, 128), cmap),
            pl.BlockSpec((128, 128), cmap),
            pl.BlockSpec((128, 128), cmap),
            pl.BlockSpec((1, 144), cmap),
            pl.BlockSpec((1, 144), cmap),
            pl.BlockSpec((1, 4), cmap),
        ],
        out_specs=pl.BlockSpec((Bb, 128), bmap),
        out_shape=jax.ShapeDtypeStruct((B, 128), F32),
        scratch_shapes=[
            pltpu.VMEM((1, 144), F32),
            pltpu.VMEM((1, 144), F32),
        ],
        compiler_params=pltpu.CompilerParams(
            dimension_semantics=("arbitrary",)),
    )(ci, cc, gi, gc, mask_p, wb, Wc, t36, t32, rep, rept, wa2, r4, ai, ac,
      ba1, alpha, ba2)
    return out


def _fc_kernel(Bf, ui_ref, pool_ref, ci_ref, cc_ref, xi_ref,
               w1_ref, b1_ref, a1_ref, w2_ref, b2_ref, a2_ref,
               w3_ref, b3_ref, out_ref):
    fc_in = jnp.concatenate(
        [ui_ref[...], pool_ref[...][:, 0:64], ci_ref[...], cc_ref[...],
         xi_ref[...]], axis=1)
    x = jnp.dot(fc_in, w1_ref[...], preferred_element_type=F32) + b1_ref[...]
    mean = jnp.mean(x, axis=0, keepdims=True)
    var = jnp.mean((x - mean) ** 2, axis=0, keepdims=True)
    p = jax.nn.sigmoid((x - mean) / jnp.sqrt(var + EPS))
    x = x * (p + (1.0 - p) * a1_ref[...])
    x = jnp.dot(x, w2_ref[...], preferred_element_type=F32) + b2_ref[...]
    mean = jnp.mean(x, axis=0, keepdims=True)
    var = jnp.mean((x - mean) ** 2, axis=0, keepdims=True)
    p = jax.nn.sigmoid((x - mean) / jnp.sqrt(var + EPS))
    x = x * (p + (1.0 - p) * a2_ref[...])
    x = jnp.dot(x, w3_ref[...], preferred_element_type=F32) + b3_ref[...]
    m = jnp.max(x, axis=-1, keepdims=True)
    e = jnp.exp(x - m)
    out_ref[...] = e / jnp.sum(e, axis=-1, keepdims=True)


def _fc_top(pooled, ui, ci, cc, xi, W_f1, b_f1, alpha_f1, W_f2, b_f2,
            alpha_f2, W_f3, b_f3, B):
    return pl.pallas_call(
        functools.partial(_fc_kernel, B),
        out_shape=jax.ShapeDtypeStruct((B, 2), F32),
    )(ui, pooled, ci, cc, xi,
      W_f1, b_f1.reshape(1, -1), alpha_f1.reshape(1, -1),
      W_f2, b_f2.reshape(1, -1), alpha_f2.reshape(1, -1),
      W_f3, b_f3.reshape(1, -1))


def kernel(user_id, beh_item, beh_cate, cand_item, cand_cate, ctx_id, mask,
           E_user, E_item, E_cate, E_ctx,
           W_a1, b_a1, alpha_a, W_a2, b_a2,
           W_f1, b_f1, alpha_f1, W_f2, b_f2, alpha_f2, W_f3, b_f3):
    B, L = beh_item.shape
    D = E_item.shape[1]
    N = B * L
    bi = beh_item.reshape(-1).astype(jnp.int32)
    bc = beh_cate.reshape(-1).astype(jnp.int32)
    V = E_item.shape[0]
    E_item_rm, q2i = _row_majorize(E_item.T, V)
    E_item_rm = E_item_rm.reshape(4 * q2i, D)
    bi = (bi % q2i) * 4 + bi // q2i
    ct = cand_item.astype(jnp.int32)
    ct = (ct % q2i) * 4 + ct // q2i
    Vu = E_user.shape[0]
    E_user_rm, q2u = _row_majorize(E_user.T, Vu)
    E_user_rm = E_user_rm.reshape(4 * q2u, D)
    uid = user_id.astype(jnp.int32)
    uid = (uid % q2u) * 4 + uid // q2u
    gi, gc, ci, cc, ui, xi = _sc_gather_all(
        4 * q2i, E_cate.shape[0], 4 * q2u, E_ctx.shape[0],
        N, B, D,
    )(E_item_rm, E_cate, E_user_rm, E_ctx, bi, bc,
      ct, cand_cate.astype(jnp.int32),
      uid, ctx_id.astype(jnp.int32))
    gi = gi.reshape(N // 4, 128)
    gc = gc.reshape(N // 4, 128)

    mask_p = mask.reshape(N // 4, 4)
    pooled = _attention(gi, gc, ci, cc, mask_p, W_a1, b_a1, alpha_a,
                        W_a2, b_a2, B, L)
    return _fc_top(pooled, ui, ci, cc, xi, W_f1, b_f1, alpha_f1,
                   W_f2, b_f2, alpha_f2, W_f3, b_f3, B)


# speedup vs baseline: 5.9233x; 1.0052x over previous
import functools

import jax
import jax.numpy as jnp
from jax import lax
from jax.experimental import pallas as pl
from jax.experimental.pallas import tpu as pltpu
from jax.experimental.pallas import tpu_sc as plsc

EPS = 1e-08
F32 = jnp.float32


_TR_CHT = 8192


def _tr_kernel(b0, b1, b2, b3, eye_ref, out_ref):
    b = jnp.concatenate([b0[...], b1[...], b2[...], b3[...]], axis=0)
    out_ref[...] = lax.dot_general(b, eye_ref[...],
                                   (((0,), (0,)), ((), ())),
                                   preferred_element_type=F32)


def _row_majorize(tabT, V):
    nmax = (V - 1) // _TR_CHT
    ni = -(-(V // 4) // _TR_CHT)
    q2 = ni * _TR_CHT

    def make_map(j):
        return lambda i: (0, jnp.minimum(j * ni + i, nmax))

    out = pl.pallas_call(
        _tr_kernel,
        grid=(ni,),
        in_specs=[pl.BlockSpec((32, _TR_CHT), make_map(j)) for j in range(4)]
        + [pl.BlockSpec((128, 128), lambda i: (0, 0))],
        out_specs=pl.BlockSpec((_TR_CHT, 128), lambda i: (i, 0)),
        out_shape=jax.ShapeDtypeStruct((q2, 128), F32),
    )(tabT, tabT, tabT, tabT, jnp.eye(128, dtype=F32))
    return out, q2


def _sc_gather_all(V_item, V_cate, V_user, V_ctx, NBIG, B, D):
    info = plsc.get_sparse_core_info()
    NC, NS = info.num_cores, info.num_subcores
    NW = NC * NS
    per_w = NBIG // NW
    CH = 1600
    NCH = per_w // CH
    SB = B // NW
    P = NBIG // 4
    mesh = plsc.VectorSubcoreMesh(core_axis_name="c", subcore_axis_name="s")

    @functools.partial(
        pl.kernel,
        mesh=mesh,
        compiler_params=pltpu.CompilerParams(use_tc_tiling_on_sc=False),
        out_type=(
            jax.ShapeDtypeStruct((NBIG, D), F32),
            jax.ShapeDtypeStruct((NBIG, D), F32),
            jax.ShapeDtypeStruct((B, D), F32),
            jax.ShapeDtypeStruct((B, D), F32),
            jax.ShapeDtypeStruct((B, D), F32),
            jax.ShapeDtypeStruct((B, D), F32),
        ),
        scratch_types=[
            pltpu.VMEM((CH,), jnp.int32),
            pltpu.VMEM((CH,), jnp.int32),
            pltpu.VMEM((CH, D), F32),
            pltpu.VMEM((CH, D), F32),
            pltpu.VMEM((SB,), jnp.int32),
            pltpu.VMEM((SB, D), F32),
            pltpu.SemaphoreType.DMA,
            pltpu.SemaphoreType.DMA,
        ],
    )
    def k(Ei, Ec, Eu, Ex, bi, bc, cand_i, cand_c, uid, ctxid,
          gi_out, gc_out, ci_out, cc_out, ui_out, xi_out,
          idx0, idx1, rows0, rows1, sidx_v, srows_v, gsem, wsem):
        wid = lax.axis_index("s") * NC + lax.axis_index("c")
        idxs = (idx0, idx1)
        rows = (rows0, rows1)
        prev = None
        for tab, idxarr, outarr in ((Ei, bi, gi_out), (Ec, bc, gc_out)):
            base = wid * per_w
            for c in range(NCH):
                b = c % 2
                off = base + c * CH
                pltpu.sync_copy(idxarr.at[pl.ds(off, CH)], idxs[b])
                g = pltpu.async_copy(tab.at[idxs[b]], rows[b], gsem)
                if prev is not None:
                    prev.wait()
                g.wait()
                prev = pltpu.async_copy(rows[b], outarr.at[pl.ds(off, CH)],
                                        wsem)
        prev.wait()
        for tab, idxarr, outarr in ((Ei, cand_i, ci_out), (Ec, cand_c, cc_out),
                                    (Eu, uid, ui_out), (Ex, ctxid, xi_out)):
            off = wid * SB
            pltpu.sync_copy(idxarr.at[pl.ds(off, SB)], sidx_v)
            pltpu.async_copy(tab.at[sidx_v], srows_v, gsem).wait()
            pltpu.sync_copy(srows_v, outarr.at[pl.ds(off, SB)])

    return k


def _attn_kernel(NB, Bb, RB, N, ci_ref, cc_ref, gi_ref, gc_ref, mask_ref,
                 wb_ref, wc_ref, t36_ref, t32_ref, rep_ref, rept_ref,
                 wa2_ref, r4_ref, ai_ref, ac_ref, ba1_ref, alpha_ref, ba2_ref,
                 out_ref, ssum, ssq):
    i = pl.program_id(0)

    ci = ci_ref[...]
    cc = cc_ref[...]
    gi = gi_ref[...]
    gc = gc_ref[...]
    rep = rep_ref[...]
    wc = wc_ref[...]

    q = jnp.dot(ci, wc[0:32, :], preferred_element_type=F32)
    q = q + jnp.dot(cc, wc[32:64, :], preferred_element_type=F32)
    q144 = jnp.dot(q, t36_ref[...], preferred_element_type=F32)
    qrows = jnp.dot(rep, q144, preferred_element_type=F32)
    ci4 = jnp.dot(ci, t32_ref[...], preferred_element_type=F32)
    cc4 = jnp.dot(cc, t32_ref[...], preferred_element_type=F32)
    cirows = jnp.dot(rep, ci4, preferred_element_type=F32)
    ccrows = jnp.dot(rep, cc4, preferred_element_type=F32)
    prodi = gi * cirows
    prodc = gc * ccrows
    BF = jnp.bfloat16
    x = jnp.concatenate(
        [gi.astype(BF), gc.astype(BF), prodi.astype(BF), prodc.astype(BF)],
        axis=1)
    h = jnp.dot(x, wb_ref[...].astype(BF), preferred_element_type=F32)
    h = h + qrows + ba1_ref[...]

    @pl.when(i == 0)
    def _init():
        ssum[...] = jnp.zeros_like(ssum)
        ssq[...] = jnp.zeros_like(ssq)

    @pl.when(i < NB)
    def _sweep1():
        ssum[...] += jnp.sum(h, axis=0, keepdims=True)
        ssq[...] += jnp.sum(h * h, axis=0, keepdims=True)
        out_ref[...] = jnp.zeros_like(out_ref)

    @pl.when(i >= NB)
    def _sweep2():
        t36 = t36_ref[...]
        s36 = jnp.dot(ssum[...], t36.T, preferred_element_type=F32)
        e36 = jnp.dot(ssq[...], t36.T, preferred_element_type=F32)
        m36 = s36 / N
        v36 = e36 / N - m36 * m36
        m144 = jnp.dot(m36, t36, preferred_element_type=F32)
        v144 = jnp.dot(v36, t36, preferred_element_type=F32)
        p = jax.nn.sigmoid((h - m144) * lax.rsqrt(v144 + EPS))
        d = h * (p + (1.0 - p) * alpha_ref[...])
        w4 = jnp.dot(d, wa2_ref[...], preferred_element_type=F32) + ba2_ref[...]
        w4 = w4 * mask_ref[...]
        wbig = jnp.dot(w4, r4_ref[...], preferred_element_type=F32)
        rept = rept_ref[...]
        si = jnp.dot(rept, wbig * gi, preferred_element_type=F32)
        sc2 = jnp.dot(rept, wbig * gc, preferred_element_type=F32)
        out_ref[...] = (jnp.dot(si, ai_ref[...], preferred_element_type=F32)
                        + jnp.dot(sc2, ac_ref[...], preferred_element_type=F32))


def _attention(gi, gc, ci, cc, mask_p, W_a1, b_a1, alpha_a, W_a2, b_a2,
               B, L):
    NB = 16
    Bb = B // NB
    RB = Bb * (L // 4)
    P = B * L // 4
    N = B * L
    eye4 = jnp.eye(4, dtype=F32)
    W1, W2, W3, W4 = (W_a1[0:64], W_a1[64:128], W_a1[128:192], W_a1[192:256])
    Wi = W1 - W3
    Wc = W2 + W3
    wb = jnp.concatenate([
        jnp.kron(eye4, Wi[0:32]), jnp.kron(eye4, Wi[32:64]),
        jnp.kron(eye4, W4[0:32]), jnp.kron(eye4, W4[32:64])], axis=0)
    t36 = jnp.tile(jnp.eye(36, dtype=F32), (1, 4))
    t32 = jnp.tile(jnp.eye(32, dtype=F32), (1, 4))
    rep = jnp.kron(jnp.eye(Bb, dtype=F32), jnp.ones((L // 4, 1), F32))
    rept = rep.T
    wa2 = jnp.kron(eye4, W_a2)
    r4 = jnp.kron(eye4, jnp.ones((1, 32), F32))
    t32v = jnp.tile(jnp.eye(32, dtype=F32), (4, 1))
    ai = jnp.concatenate([t32v, jnp.zeros((128, 96), F32)], axis=1)
    ac = jnp.concatenate([jnp.zeros((128, 32), F32), t32v,
                          jnp.zeros((128, 64), F32)], axis=1)
    ba1 = jnp.tile(b_a1.reshape(1, 36), (1, 4))
    alpha = jnp.tile(alpha_a.reshape(1, 36), (1, 4))
    ba2 = jnp.tile(b_a2.reshape(1, 1), (1, 4))

    def cmap(i):
        return (0, 0)

    def bmap(i):
        return (i % NB, 0)

    grid = (2 * NB,)
    out = pl.pallas_call(
        functools.partial(_attn_kernel, NB, Bb, RB, float(N)),
        grid=grid,
        in_specs=[
            pl.BlockSpec((Bb, 32), bmap),
            pl.BlockSpec((Bb, 32), bmap),
            pl.BlockSpec((RB, 128), bmap),
            pl.BlockSpec((RB, 128), bmap),
            pl.BlockSpec((RB, 4), bmap),
            pl.BlockSpec((512, 144), cmap),
            pl.BlockSpec((64, 36), cmap),
            pl.BlockSpec((36, 144), cmap),
            pl.BlockSpec((32, 128), cmap),
            pl.BlockSpec((RB, Bb), cmap),
            pl.BlockSpec((Bb, RB), cmap),
            pl.BlockSpec((144, 4), cmap),
            pl.BlockSpec((4, 128), cmap),
            pl.BlockSpec((128, 128), cmap),
            pl.BlockSpec((128, 128), cmap),
            pl.BlockSpec((1, 144), cmap),
            pl.BlockSpec((1, 144), cmap),
            pl.BlockSpec((1, 4), cmap),
        ],
        out_specs=pl.BlockSpec((Bb, 128), bmap),
        out_shape=jax.ShapeDtypeStruct((B, 128), F32),
        scratch_shapes=[
            pltpu.VMEM((1, 144), F32),
            pltpu.VMEM((1, 144), F32),
        ],
        compiler_params=pltpu.CompilerParams(
            dimension_semantics=("arbitrary",)),
    )(ci, cc, gi, gc, mask_p, wb, Wc, t36, t32, rep, rept, wa2, r4, ai, ac,
      ba1, alpha, ba2)
    return out


def _fc_kernel(Bf, ui_ref, pool_ref, ci_ref, cc_ref, xi_ref,
               w1_ref, b1_ref, a1_ref, w2_ref, b2_ref, a2_ref,
               w3_ref, b3_ref, out_ref):
    fc_in = jnp.concatenate(
        [ui_ref[...], pool_ref[...][:, 0:64], ci_ref[...], cc_ref[...],
         xi_ref[...]], axis=1)
    x = jnp.dot(fc_in, w1_ref[...], preferred_element_type=F32) + b1_ref[...]
    mean = jnp.mean(x, axis=0, keepdims=True)
    var = jnp.mean((x - mean) ** 2, axis=0, keepdims=True)
    p = jax.nn.sigmoid((x - mean) / jnp.sqrt(var + EPS))
    x = x * (p + (1.0 - p) * a1_ref[...])
    x = jnp.dot(x, w2_ref[...], preferred_element_type=F32) + b2_ref[...]
    mean = jnp.mean(x, axis=0, keepdims=True)
    var = jnp.mean((x - mean) ** 2, axis=0, keepdims=True)
    p = jax.nn.sigmoid((x - mean) / jnp.sqrt(var + EPS))
    x = x * (p + (1.0 - p) * a2_ref[...])
    x = jnp.dot(x, w3_ref[...], preferred_element_type=F32) + b3_ref[...]
    m = jnp.max(x, axis=-1, keepdims=True)
    e = jnp.exp(x - m)
    out_ref[...] = e / jnp.sum(e, axis=-1, keepdims=True)


def _fc_top(pooled, ui, ci, cc, xi, W_f1, b_f1, alpha_f1, W_f2, b_f2,
            alpha_f2, W_f3, b_f3, B):
    return pl.pallas_call(
        functools.partial(_fc_kernel, B),
        out_shape=jax.ShapeDtypeStruct((B, 2), F32),
    )(ui, pooled, ci, cc, xi,
      W_f1, b_f1.reshape(1, -1), alpha_f1.reshape(1, -1),
      W_f2, b_f2.reshape(1, -1), alpha_f2.reshape(1, -1),
      W_f3, b_f3.reshape(1, -1))


def kernel(user_id, beh_item, beh_cate, cand_item, cand_cate, ctx_id, mask,
           E_user, E_item, E_cate, E_ctx,
           W_a1, b_a1, alpha_a, W_a2, b_a2,
           W_f1, b_f1, alpha_f1, W_f2, b_f2, alpha_f2, W_f3, b_f3):
    B, L = beh_item.shape
    D = E_item.shape[1]
    N = B * L
    bi = beh_item.reshape(-1).astype(jnp.int32)
    bc = beh_cate.reshape(-1).astype(jnp.int32)
    V = E_item.shape[0]
    E_item_rm, q2i = _row_majorize(E_item.T, V)
    E_item_rm = E_item_rm.reshape(4 * q2i, D)
    bi = (bi % q2i) * 4 + bi // q2i
    ct = cand_item.astype(jnp.int32)
    ct = (ct % q2i) * 4 + ct // q2i
    Vu = E_user.shape[0]
    E_user_rm, q2u = _row_majorize(E_user.T, Vu)
    E_user_rm = E_user_rm.reshape(4 * q2u, D)
    uid = user_id.astype(jnp.int32)
    uid = (uid % q2u) * 4 + uid // q2u
    gi, gc, ci, cc, ui, xi = _sc_gather_all(
        4 * q2i, E_cate.shape[0], 4 * q2u, E_ctx.shape[0],
        N, B, D,
    )(E_item_rm, E_cate, E_user_rm, E_ctx, bi, bc,
      ct, cand_cate.astype(jnp.int32),
      uid, ctx_id.astype(jnp.int32))
    gi = gi.reshape(N // 4, 128)
    gc = gc.reshape(N // 4, 128)

    mask_p = mask.reshape(N // 4, 4)
    pooled = _attention(gi, gc, ci, cc, mask_p, W_a1, b_a1, alpha_a,
                        W_a2, b_a2, B, L)
    return _fc_top(pooled, ui, ci, cc, xi, W_f1, b_f1, alpha_f1,
                   W_f2, b_f2, alpha_f2, W_f3, b_f3, B)
